# Initial kernel scaffold; baseline (speedup 1.0000x reference)
import jax
import jax.numpy as jnp
from jax.experimental import pallas as pl


def kernel(u_pos, v_pos, v_neg, batch_size, U_emb, V_emb):
    raise NotImplementedError("write your pallas kernel here")


# speedup vs baseline: 4.3604x; 4.3604x over previous
import functools

import jax
import jax.numpy as jnp
from jax import lax
from jax.experimental import pallas as pl
from jax.experimental.pallas import tpu as pltpu
from jax.experimental.pallas import tpu_sc as plsc

B = 16384
D = 64
NNEG = 20
NC = 2
NS = 16
NW = NC * NS
BPW = B // NW
CH = 32
NCH = BPW // CH
NIDX_ROWS = BPW * NNEG // 128
NEG_DMA = CH * NNEG // 128
VREGS = D // 16


def _sc_body(uidx_h, vidx_h, nidx_h, U_h, V_h, op_h, on_h,
             uidx, vidx, nidx, ubuf, vbuf, nbuf, opb, onb, sem):
    wid = lax.axis_index("c") * NS + lax.axis_index("s")
    pltpu.sync_copy(uidx_h.at[wid], uidx)
    pltpu.sync_copy(vidx_h.at[wid], vidx)
    pltpu.sync_copy(nidx_h.at[wid], nidx)
    iota = lax.iota(jnp.int32, 16)
    zero = jnp.zeros((16,), jnp.float32)

    def chunk(c, carry):
        hu = pltpu.async_copy(U_h.at[uidx.at[c]], ubuf, sem)
        hv = pltpu.async_copy(V_h.at[vidx.at[c]], vbuf, sem)
        hn = [pltpu.async_copy(V_h.at[nidx.at[NEG_DMA * c + j]],
                               nbuf.at[pl.ds(j * 128, 128)], sem)
              for j in range(NEG_DMA)]
        hu.wait()
        hv.wait()
        for h in hn:
            h.wait()

        for g in range(CH // 16):
            rows16 = g * 16 + iota
            nrows = rows16 * NNEG

            def dim_body(d, acc):
                ap, an = acc
                dcol = jnp.full((16,), d, jnp.int32)
                ug = plsc.load_gather(ubuf, [rows16, dcol])
                vg = plsc.load_gather(vbuf, [rows16, dcol])
                ns = plsc.load_gather(nbuf, [nrows, dcol])
                for k in range(1, NNEG):
                    ns = ns + plsc.load_gather(nbuf, [nrows + k, dcol])
                return (ap + ug * vg, an + ug * ns)

            ap, an = lax.fori_loop(0, D, dim_body, (zero, zero))
            opb[pl.ds(c * CH + g * 16, 16)] = ap
            onb[pl.ds(c * CH + g * 16, 16)] = an
        return carry

    lax.fori_loop(0, NCH, chunk, 0)
    pltpu.sync_copy(opb, op_h.at[pl.ds(wid * BPW, BPW)])
    pltpu.sync_copy(onb, on_h.at[pl.ds(wid * BPW, BPW)])


def _tc_loss_body(sp_ref, sn_ref, o_ref):
    x = sp_ref[...]
    y = sn_ref[...]

    def ls(t):
        return jnp.minimum(t, 0.0) - jnp.log1p(jnp.exp(-jnp.abs(t)))

    o_ref[0, 0] = -jnp.sum(ls(x) + ls(-y)) * (1.0 / B)


@jax.jit
def _w2v_loss(u_idx, v_idx, n_idx, U_emb, V_emb):
    mesh = plsc.VectorSubcoreMesh(core_axis_name="c", subcore_axis_name="s")
    sc = pl.kernel(
        _sc_body,
        out_type=[
            jax.ShapeDtypeStruct((B,), jnp.float32),
            jax.ShapeDtypeStruct((B,), jnp.float32),
        ],
        mesh=mesh,
        compiler_params=pltpu.CompilerParams(
            needs_layout_passes=False, use_tc_tiling_on_sc=False
        ),
        scratch_types=[
            pltpu.VMEM((NCH, CH), jnp.int32),
            pltpu.VMEM((NCH, CH), jnp.int32),
            pltpu.VMEM((NIDX_ROWS, 128), jnp.int32),
            pltpu.VMEM((CH, D), jnp.float32),
            pltpu.VMEM((CH, D), jnp.float32),
            pltpu.VMEM((CH * NNEG, D), jnp.float32),
            pltpu.VMEM((BPW,), jnp.float32),
            pltpu.VMEM((BPW,), jnp.float32),
            pltpu.SemaphoreType.DMA,
        ],
    )
    s_pos, s_neg = sc(u_idx, v_idx, n_idx, U_emb, V_emb)
    out = pl.pallas_call(
        _tc_loss_body,
        out_shape=jax.ShapeDtypeStruct((1, 1), jnp.float32),
        out_specs=pl.BlockSpec(memory_space=pltpu.SMEM),
    )(s_pos.reshape(128, 128), s_neg.reshape(128, 128))
    return out[0, 0]


def kernel(u_pos, v_pos, v_neg, batch_size, U_emb, V_emb):
    u_idx = u_pos.reshape(NW, NCH, CH)
    v_idx = v_pos.reshape(NW, NCH, CH)
    n_idx = v_neg.reshape(NW, NIDX_ROWS, 128)
    return _w2v_loss(u_idx, v_idx, n_idx, U_emb, V_emb)
